# Initial kernel scaffold; baseline (speedup 1.0000x reference)
import jax
import jax.numpy as jnp
from jax.experimental import pallas as pl


def kernel(x, edge_index, Wl0, bl0, Wr0, Wlin0, blin0, Wl1, bl1, Wr1, Wlin1, blin1, Wl2, bl2, Wr2, Wlin2, blin2):
    raise NotImplementedError("write your pallas kernel here")


# speedup vs baseline: 33.8854x; 33.8854x over previous
import functools

import numpy as np
import jax
import jax.numpy as jnp
from jax import lax
from jax.experimental import pallas as pl
from jax.experimental.pallas import tpu as pltpu
from jax.experimental.pallas import tpu_sc as plsc

_N = 100000
_D = 16
_NC = 2
_NS = 16
_NW = _NC * _NS
_SUB = 128
_K = 8
_CH = _SUB * _K
_RPS = 6272
_NACC = _RPS * _NS


def _sc_scatter_fn(epw, with_counts):
    nchunk = epw // _CH
    mesh = plsc.VectorSubcoreMesh(core_axis_name="c", subcore_axis_name="s")
    out_type = [jax.ShapeDtypeStruct((_NC, _NACC, _D), jnp.float32)]
    if with_counts:
        out_type.append(jax.ShapeDtypeStruct((_NC, _NACC), jnp.float32))
    scratch = [
        pltpu.VMEM_SHARED((_NACC, _D), jnp.float32),
        pltpu.VMEM((_K, _SUB), jnp.int32),
        pltpu.VMEM((_K, _SUB), jnp.int32),
        pltpu.VMEM((_K, _SUB, _D), jnp.float32),
        pltpu.SemaphoreType.DMA,
    ]
    if with_counts:
        scratch += [
            pltpu.VMEM_SHARED((_NACC,), jnp.float32),
            pltpu.VMEM((1, _SUB), jnp.float32),
        ]

    def body(x_h, src_h, dst_h, z2_h, z1_h, ones_h, *rest):
        if with_counts:
            out_h, cnt_h, acc, src_v, dst_v, rows_v, sem, cacc, ones_v = rest
        else:
            out_h, acc, src_v, dst_v, rows_v, sem = rest
        c = lax.axis_index("c")
        s = lax.axis_index("s")
        g = c * _NS + s
        r0 = s * _RPS
        pltpu.sync_copy(z2_h.at[pl.ds(r0, _RPS)], acc.at[pl.ds(r0, _RPS)])
        if with_counts:
            pltpu.sync_copy(z1_h.at[pl.ds(r0, _RPS)], cacc.at[pl.ds(r0, _RPS)])
            pltpu.sync_copy(ones_h, ones_v)
        plsc.subcore_barrier()

        rows_per_chunk = _CH // _SUB
        row_base0 = g * (epw // _SUB)

        def chunk(i, carry):
            row0 = row_base0 + i * rows_per_chunk
            pltpu.sync_copy(src_h.at[pl.ds(row0, _K)], src_v)
            pltpu.sync_copy(dst_h.at[pl.ds(row0, _K)], dst_v)
            descs = [
                pltpu.async_copy(x_h.at[src_v.at[j]], rows_v.at[j], sem)
                for j in range(_K)
            ]
            for d in descs:
                d.wait()
            for j in range(_K):
                pltpu.sync_copy(rows_v.at[j], acc.at[dst_v.at[j]], add=True)
                if with_counts:
                    pltpu.sync_copy(ones_v.at[0], cacc.at[dst_v.at[j]], add=True)
            return carry

        lax.fori_loop(0, nchunk, chunk, 0)
        plsc.subcore_barrier()
        pltpu.sync_copy(acc.at[pl.ds(r0, _RPS)], out_h.at[c, pl.ds(r0, _RPS)])
        if with_counts:
            pltpu.sync_copy(cacc.at[pl.ds(r0, _RPS)], cnt_h.at[c, pl.ds(r0, _RPS)])

    return pl.kernel(body, out_type=tuple(out_type), mesh=mesh,
                     scratch_types=scratch,
                     compiler_params=pltpu.CompilerParams(
                         use_tc_tiling_on_sc=False))


def _dense_body(p_ref, x_ref, c8_ref, s_ref, wl_ref, wr_ref, b_ref, o_ref):
    hi = jax.lax.Precision.HIGHEST
    c0 = c8_ref[0]
    c1 = c8_ref[1]
    cexp = jnp.dot(c0 + c1, s_ref[...], precision=hi,
                   preferred_element_type=jnp.float32)
    agg = (p_ref[0] + p_ref[1]) / jnp.maximum(cexp, 1.0)
    o_ref[...] = (
        jnp.dot(agg, wl_ref[...], precision=hi, preferred_element_type=jnp.float32)
        + jnp.dot(x_ref[...], wr_ref[...], precision=hi,
                  preferred_element_type=jnp.float32)
        + b_ref[...]
    )


_ROWS = _NACC * _D // 128
_BR = 1568


def _dense(p, xr, c8, smat, wlb, wrb, bias):
    grid = (_ROWS // _BR,)
    return pl.pallas_call(
        _dense_body,
        grid=grid,
        in_specs=[
            pl.BlockSpec((_NC, _BR, 128), lambda i: (0, i, 0)),
            pl.BlockSpec((_BR, 128), lambda i: (i, 0)),
            pl.BlockSpec((_NC, _BR, 8), lambda i: (0, i, 0)),
            pl.BlockSpec((8, 128), lambda i: (0, 0)),
            pl.BlockSpec((128, 128), lambda i: (0, 0)),
            pl.BlockSpec((128, 128), lambda i: (0, 0)),
            pl.BlockSpec((1, 128), lambda i: (0, 0)),
        ],
        out_specs=pl.BlockSpec((_BR, 128), lambda i: (i, 0)),
        out_shape=jax.ShapeDtypeStruct((_ROWS, 128), jnp.float32),
    )(p, xr, c8, smat, wlb, wrb, bias)


def kernel(x, edge_index, Wl0, bl0, Wr0, Wlin0, blin0, Wl1, bl1, Wr1, Wlin1,
           blin1, Wl2, bl2, Wr2, Wlin2, blin2):
    src = edge_index[0].astype(jnp.int32)
    dst = edge_index[1].astype(jnp.int32)
    e = src.shape[0]
    epw = ((e + _NW * _CH - 1) // (_NW * _CH)) * _CH
    e_pad = epw * _NW
    src_p = jnp.concatenate([src, jnp.zeros((e_pad - e,), jnp.int32)])
    dst_p = jnp.concatenate([dst, jnp.full((e_pad - e,), _N, jnp.int32)])
    src2 = src_p.reshape(-1, _SUB)
    dst2 = dst_p.reshape(-1, _SUB)
    z2 = jnp.zeros((_NACC, _D), jnp.float32)
    z1 = jnp.zeros((_NACC,), jnp.float32)
    ones = jnp.ones((1, _SUB), jnp.float32)

    eye8 = jnp.eye(8, dtype=jnp.float32)
    smat_np = np.zeros((8, 128), np.float32)
    for i in range(8):
        smat_np[i, 16 * i:16 * (i + 1)] = 1.0
    smat = jnp.asarray(smat_np)

    params = [(Wl0, bl0, Wr0, Wlin0, blin0), (Wl1, bl1, Wr1, Wlin1, blin1),
              (Wl2, bl2, Wr2, Wlin2, blin2)]

    sc_first = _sc_scatter_fn(epw, True)
    sc_rest = _sc_scatter_fn(epw, False)

    xl = jnp.concatenate([x, jnp.zeros((_NACC - _N, _D), jnp.float32)])
    c8 = None
    for li, (Wl, bl, Wr, Wlin, blin) in enumerate(params):
        if li == 0:
            p, cnt = sc_first(xl, src2, dst2, z2, z1, ones)
            c8 = cnt.reshape(_NC, _ROWS, 8)
        else:
            (p,) = sc_rest(xl, src2, dst2, z2, z1, ones)
        wlb = jnp.kron(eye8, Wl)
        wrb = jnp.kron(eye8, Wr + Wlin)
        bias = jnp.tile(bl + blin, 8).reshape(1, 128)
        xr = xl.reshape(_ROWS, 128)
        pr = p.reshape(_NC, _ROWS, 128)
        xl = _dense(pr, xr, c8, smat, wlb, wrb, bias).reshape(_NACC, _D)
    return xl[:_N]


# speedup vs baseline: 39.6553x; 1.1703x over previous
import functools

import numpy as np
import jax
import jax.numpy as jnp
from jax import lax
from jax.experimental import pallas as pl
from jax.experimental.pallas import tpu as pltpu
from jax.experimental.pallas import tpu_sc as plsc

_N = 100000
_D = 16
_NC = 2
_NS = 16
_NW = _NC * _NS
_SUB = 128
_RD = 4
_KS = 3
_KC = 8
_RPS = 6272
_NACC = _RPS * _NS

_SC_PARAMS = pltpu.CompilerParams(use_tc_tiling_on_sc=False)


def _pad_epw(e, k):
    step = _RD * k * _SUB
    return max(2 * step, ((e + _NW * step - 1) // (_NW * step)) * step)


def _sum_kernel(epw):
    k = _KS
    ch = k * _SUB
    nchunk = epw // ch
    assert nchunk % _RD == 0 and nchunk >= 2 * _RD
    mesh = plsc.VectorSubcoreMesh(core_axis_name="c", subcore_axis_name="s")

    def body(x_h, src_h, dst_h, z2_h, out_h, acc, src_v, dst_v, rows_v,
             isem, gsem, ssem):
        c = lax.axis_index("c")
        s = lax.axis_index("s")
        w = c * _NS + s
        r0 = s * _RPS
        wrow = w * (epw // _SUB)

        def fire_idx(g, b):
            pltpu.async_copy(src_h.at[pl.ds(wrow + g * k, k)], src_v[b],
                             isem[b])
            pltpu.async_copy(dst_h.at[pl.ds(wrow + g * k, k)], dst_v[b],
                             isem[b])

        def wait_idx(b):
            pltpu.make_async_copy(src_h.at[pl.ds(0, k)], src_v[b],
                                  isem[b]).wait()
            pltpu.make_async_copy(src_h.at[pl.ds(0, k)], dst_v[b],
                                  isem[b]).wait()

        def fire_gathers(b):
            for j in range(k):
                pltpu.async_copy(x_h.at[src_v[b].at[j]], rows_v[b].at[j],
                                 gsem[b])

        def wait_gathers(b):
            for j in range(k):
                pltpu.make_async_copy(x_h.at[pl.ds(0, _SUB)],
                                      rows_v[b].at[j], gsem[b]).wait()

        def fire_scatters(b):
            for j in range(k):
                pltpu.async_copy(rows_v[b].at[j], acc.at[dst_v[b].at[j]],
                                 ssem[b], add=True)

        def wait_scatters(b):
            for j in range(k):
                pltpu.make_async_copy(rows_v[b].at[j],
                                      acc.at[pl.ds(0, _SUB)], ssem[b]).wait()

        pltpu.sync_copy(z2_h.at[pl.ds(r0, _RPS)], acc.at[pl.ds(r0, _RPS)])
        plsc.subcore_barrier()

        fire_idx(0, 0)
        fire_idx(1, 1)
        fire_idx(2, 2)
        wait_idx(0)
        fire_gathers(0)
        fire_idx(3, 3)
        wait_idx(1)
        fire_gathers(1)
        wait_gathers(0)
        fire_scatters(0)

        def blk(i0, carry):
            for r in range(_RD):
                g = 2 + i0 * _RD + r
                wait_scatters(r)
                fire_idx(g + 2, r)
                wait_idx((2 + r) % _RD)
                fire_gathers((2 + r) % _RD)
                wait_gathers((1 + r) % _RD)
                fire_scatters((1 + r) % _RD)
            return carry

        lax.fori_loop(0, (nchunk - 4) // _RD, blk, 0)

        wait_scatters(0)
        wait_idx(2)
        fire_gathers(2)
        wait_gathers(1)
        fire_scatters(1)
        wait_scatters(1)
        wait_idx(3)
        fire_gathers(3)
        wait_gathers(2)
        fire_scatters(2)
        wait_gathers(3)
        fire_scatters(3)
        wait_scatters(2)
        wait_scatters(3)

        plsc.subcore_barrier()
        pltpu.sync_copy(acc.at[pl.ds(r0, _RPS)], out_h.at[c, pl.ds(r0, _RPS)])

    return pl.kernel(
        body,
        out_type=jax.ShapeDtypeStruct((_NC, _NACC, _D), jnp.float32),
        mesh=mesh,
        scratch_types=[
            pltpu.VMEM_SHARED((_NACC, _D), jnp.float32),
            [pltpu.VMEM((k, _SUB), jnp.int32) for _ in range(_RD)],
            [pltpu.VMEM((k, _SUB), jnp.int32) for _ in range(_RD)],
            [pltpu.VMEM((k, _SUB, _D), jnp.float32) for _ in range(_RD)],
            [pltpu.SemaphoreType.DMA for _ in range(_RD)],
            [pltpu.SemaphoreType.DMA for _ in range(_RD)],
            [pltpu.SemaphoreType.DMA for _ in range(_RD)],
        ],
        compiler_params=_SC_PARAMS,
    )


def _count_kernel(epw):
    k = _KC
    ch = k * _SUB
    nchunk = epw // ch
    assert nchunk % _RD == 0 and nchunk >= 2 * _RD
    mesh = plsc.VectorSubcoreMesh(core_axis_name="c", subcore_axis_name="s")

    def body(dst_h, z1_h, ones_h, cnt_h, cacc, dst_v, ones_v, isem, ssem):
        c = lax.axis_index("c")
        s = lax.axis_index("s")
        w = c * _NS + s
        r0 = s * _RPS
        wrow = w * (epw // _SUB)

        def fire_idx(g, b):
            pltpu.async_copy(dst_h.at[pl.ds(wrow + g * k, k)], dst_v[b],
                             isem[b])

        def wait_idx(b):
            pltpu.make_async_copy(dst_h.at[pl.ds(0, k)], dst_v[b],
                                  isem[b]).wait()

        def fire_scatters(b):
            for j in range(k):
                pltpu.async_copy(ones_v.at[0], cacc.at[dst_v[b].at[j]],
                                 ssem[b], add=True)

        def wait_scatters(b):
            for j in range(k):
                pltpu.make_async_copy(ones_v.at[0], cacc.at[pl.ds(0, _SUB)],
                                      ssem[b]).wait()

        pltpu.sync_copy(z1_h.at[pl.ds(r0, _RPS)], cacc.at[pl.ds(r0, _RPS)])
        pltpu.sync_copy(ones_h, ones_v)
        plsc.subcore_barrier()

        fire_idx(0, 0)
        fire_idx(1, 1)
        fire_idx(2, 2)
        wait_idx(0)
        fire_scatters(0)
        fire_idx(3, 3)
        wait_idx(1)
        fire_scatters(1)

        def blk(i0, carry):
            for r in range(_RD):
                g = 2 + i0 * _RD + r
                wait_scatters(r)
                fire_idx(g + 2, r)
                wait_idx((2 + r) % _RD)
                fire_scatters((2 + r) % _RD)
            return carry

        lax.fori_loop(0, (nchunk - 4) // _RD, blk, 0)

        wait_scatters(0)
        wait_idx(2)
        fire_scatters(2)
        wait_scatters(1)
        wait_idx(3)
        fire_scatters(3)
        wait_scatters(2)
        wait_scatters(3)

        plsc.subcore_barrier()
        pltpu.sync_copy(cacc.at[pl.ds(r0, _RPS)], cnt_h.at[c, pl.ds(r0, _RPS)])

    return pl.kernel(
        body,
        out_type=jax.ShapeDtypeStruct((_NC, _NACC), jnp.float32),
        mesh=mesh,
        scratch_types=[
            pltpu.VMEM_SHARED((_NACC,), jnp.float32),
            [pltpu.VMEM((k, _SUB), jnp.int32) for _ in range(_RD)],
            pltpu.VMEM((1, _SUB), jnp.float32),
            [pltpu.SemaphoreType.DMA for _ in range(_RD)],
            [pltpu.SemaphoreType.DMA for _ in range(_RD)],
        ],
        compiler_params=_SC_PARAMS,
    )


def _dense_body(p_ref, x_ref, c8_ref, s_ref, wl_ref, wr_ref, b_ref, o_ref):
    hi = jax.lax.Precision.HIGHEST
    cexp = jnp.dot(c8_ref[0] + c8_ref[1], s_ref[...], precision=hi,
                   preferred_element_type=jnp.float32)
    agg = (p_ref[0] + p_ref[1]) / jnp.maximum(cexp, 1.0)
    o_ref[...] = (
        jnp.dot(agg, wl_ref[...], precision=hi, preferred_element_type=jnp.float32)
        + jnp.dot(x_ref[...], wr_ref[...], precision=hi,
                  preferred_element_type=jnp.float32)
        + b_ref[...]
    )


_ROWS = _NACC * _D // 128
_BR = 1568


def _dense(p, xr, c8, smat, wlb, wrb, bias):
    grid = (_ROWS // _BR,)
    return pl.pallas_call(
        _dense_body,
        grid=grid,
        in_specs=[
            pl.BlockSpec((_NC, _BR, 128), lambda i: (0, i, 0)),
            pl.BlockSpec((_BR, 128), lambda i: (i, 0)),
            pl.BlockSpec((_NC, _BR, 8), lambda i: (0, i, 0)),
            pl.BlockSpec((8, 128), lambda i: (0, 0)),
            pl.BlockSpec((128, 128), lambda i: (0, 0)),
            pl.BlockSpec((128, 128), lambda i: (0, 0)),
            pl.BlockSpec((1, 128), lambda i: (0, 0)),
        ],
        out_specs=pl.BlockSpec((_BR, 128), lambda i: (i, 0)),
        out_shape=jax.ShapeDtypeStruct((_ROWS, 128), jnp.float32),
    )(p, xr, c8, smat, wlb, wrb, bias)


def _pad_edges(a, epw, fill):
    e = a.shape[0]
    return jnp.concatenate(
        [a, jnp.full((epw * _NW - e,), fill, jnp.int32)]).reshape(-1, _SUB)


def kernel(x, edge_index, Wl0, bl0, Wr0, Wlin0, blin0, Wl1, bl1, Wr1, Wlin1,
           blin1, Wl2, bl2, Wr2, Wlin2, blin2):
    src = edge_index[0].astype(jnp.int32)
    dst = edge_index[1].astype(jnp.int32)
    e = src.shape[0]
    epw_s = _pad_epw(e, _KS)
    src2 = _pad_edges(src, epw_s, 0)
    dst2 = _pad_edges(dst, epw_s, _N)
    epw_c = _pad_epw(e, _KC)
    dst2c = _pad_edges(dst, epw_c, _N)

    z2 = jnp.zeros((_NACC, _D), jnp.float32)
    z1 = jnp.zeros((_NACC,), jnp.float32)
    ones = jnp.ones((1, _SUB), jnp.float32)

    eye8 = jnp.eye(8, dtype=jnp.float32)
    smat_np = np.zeros((8, 128), np.float32)
    for i in range(8):
        smat_np[i, 16 * i:16 * (i + 1)] = 1.0
    smat = jnp.asarray(smat_np)

    params = [(Wl0, bl0, Wr0, Wlin0, blin0), (Wl1, bl1, Wr1, Wlin1, blin1),
              (Wl2, bl2, Wr2, Wlin2, blin2)]

    sc_sum = _sum_kernel(epw_s)
    cnt = _count_kernel(epw_c)(dst2c, z1, ones)
    c8 = cnt.reshape(_NC, _ROWS, 8)

    xl = jnp.concatenate([x, jnp.zeros((_NACC - _N, _D), jnp.float32)])
    for (Wl, bl, Wr, Wlin, blin) in params:
        p = sc_sum(xl, src2, dst2, z2)
        wlb = jnp.kron(eye8, Wl)
        wrb = jnp.kron(eye8, Wr + Wlin)
        bias = jnp.tile(bl + blin, 8).reshape(1, 128)
        xr = xl.reshape(_ROWS, 128)
        pr = p.reshape(_NC, _ROWS, 128)
        xl = _dense(pr, xr, c8, smat, wlb, wrb, bias).reshape(_NACC, _D)
    return xl[:_N]
